# Initial kernel scaffold; baseline (speedup 1.0000x reference)
import jax
import jax.numpy as jnp
from jax.experimental import pallas as pl


def kernel(graph_features, W1, b1, W2, b2, Wp, bp):
    raise NotImplementedError("write your pallas kernel here")


# speedup vs baseline: 3.2509x; 3.2509x over previous
import functools

import jax
import jax.numpy as jnp
from jax import lax
from jax.experimental import pallas as pl
from jax.experimental.pallas import tpu as pltpu
from jax.experimental.pallas import tpu_sc as plsc

_NUM_NODES = 32
_NODE_DIM = 64
_K = 8
_PAIR = 2 * _NODE_DIM


_ROWS_BLK = 8192
_TOK_BLK = _ROWS_BLK // _NUM_NODES


def _score_topk_body(x_ref, w1_ref, b1_ref, w2_ref, b2_ref, idx_ref):
    xb = x_ref[...]
    h = xb @ w1_ref[...] + b1_ref[...]
    h = 0.5 * h * (1.0 + lax.erf(h * (2.0 ** -0.5)))
    s = h @ w2_ref[...] + b2_ref[...]
    sc = s.reshape(_TOK_BLK, _NUM_NODES)

    lane = lax.broadcasted_iota(jnp.int32, sc.shape, 1)
    cols = []
    vals = sc
    for _ in range(_K):
        m = jnp.max(vals, axis=1, keepdims=True)
        is_max = vals >= m
        cand = jnp.where(is_max, lane, _NUM_NODES)
        k_idx = jnp.min(cand, axis=1, keepdims=True)
        cols.append(k_idx)
        vals = jnp.where(lane == k_idx, -jnp.inf, vals)
    idx = jnp.concatenate(cols, axis=1)

    tok0 = pl.program_id(0) * _TOK_BLK
    rows = tok0 + lax.broadcasted_iota(jnp.int32, (_TOK_BLK, _K), 0)
    idx_ref[...] = rows * _NUM_NODES + idx


def _score_topk(x2, W1, b1, W2, b2):
    n_rows = x2.shape[0]
    grid = n_rows // _ROWS_BLK
    return pl.pallas_call(
        _score_topk_body,
        grid=(grid,),
        in_specs=[
            pl.BlockSpec((_ROWS_BLK, _NODE_DIM), lambda i: (i, 0)),
            pl.BlockSpec((_NODE_DIM, 32), lambda i: (0, 0)),
            pl.BlockSpec((1, 32), lambda i: (0, 0)),
            pl.BlockSpec((32, 1), lambda i: (0, 0)),
            pl.BlockSpec((1, 1), lambda i: (0, 0)),
        ],
        out_specs=pl.BlockSpec((_TOK_BLK, _K), lambda i: (i, 0)),
        out_shape=jax.ShapeDtypeStruct((n_rows // _NUM_NODES, _K), jnp.int32),
    )(x2, W1, b1, W2, b2)


_CHUNK = 128


def _sc_gather(table, idx):
    info = plsc.get_sparse_core_info()
    nw = info.num_cores * info.num_subcores
    rows = idx.shape[0]
    b_per_w = rows // nw
    n_chunks = b_per_w // _CHUNK
    mesh = plsc.VectorSubcoreMesh(core_axis_name="c", subcore_axis_name="s")

    @functools.partial(
        pl.kernel,
        mesh=mesh,
        out_type=jax.ShapeDtypeStruct((rows, _PAIR), jnp.float32),
        scratch_types=[
            pltpu.VMEM((_CHUNK,), jnp.int32),
            pltpu.VMEM((_CHUNK, _PAIR), jnp.float32),
            pltpu.SemaphoreType.DMA,
        ],
    )
    def gather_kernel(table_hbm, idx_hbm, out_hbm, idx_v, rows_v, sem):
        wid = lax.axis_index("s") * info.num_cores + lax.axis_index("c")
        base = wid * b_per_w

        def body(c, _):
            off = base + c * _CHUNK
            pltpu.sync_copy(idx_hbm.at[pl.ds(off, _CHUNK)], idx_v)
            pltpu.async_copy(table_hbm.at[idx_v], rows_v, sem).wait()
            pltpu.sync_copy(rows_v, out_hbm.at[pl.ds(off, _CHUNK)])
            return 0

        lax.fori_loop(0, n_chunks, body, 0)

    return gather_kernel(table, idx)


_PROJ_BLK = 512


def _proj_body(w_ref, i_ref, wp_ref, bp_ref, o_ref):
    wide = w_ref[...]
    fidx = i_ref[...]
    parts = []
    for k in range(_K):
        low = wide[:, k * _PAIR: k * _PAIR + _NODE_DIM]
        high = wide[:, k * _PAIR + _NODE_DIM: (k + 1) * _PAIR]
        odd = (fidx[:, k: k + 1] & 1) == 1
        parts.append(jnp.where(odd, high, low))
    pooled = jnp.concatenate(parts, axis=1)
    o_ref[...] = pooled @ wp_ref[...] + bp_ref[...]


def _projection(wide, flat_idx, Wp, bp):
    n_tok = wide.shape[0]
    d = Wp.shape[1]
    grid = n_tok // _PROJ_BLK
    return pl.pallas_call(
        _proj_body,
        grid=(grid,),
        in_specs=[
            pl.BlockSpec((_PROJ_BLK, _K * _PAIR), lambda i: (i, 0)),
            pl.BlockSpec((_PROJ_BLK, _K), lambda i: (i, 0)),
            pl.BlockSpec((_K * _NODE_DIM, d), lambda i: (0, 0)),
            pl.BlockSpec((1, d), lambda i: (0, 0)),
        ],
        out_specs=pl.BlockSpec((_PROJ_BLK, d), lambda i: (i, 0)),
        out_shape=jax.ShapeDtypeStruct((n_tok, d), jnp.float32),
    )(wide, flat_idx, Wp, bp)


def kernel(graph_features, W1, b1, W2, b2, Wp, bp):
    bsz, seqlen, dmodel = graph_features.shape
    n_tok = bsz * seqlen
    x2 = graph_features.reshape(n_tok * _NUM_NODES, _NODE_DIM)

    flat_idx = _score_topk(
        x2, W1, b1.reshape(1, -1), W2, b2.reshape(1, 1)
    )

    pairs = graph_features.reshape(n_tok * _NUM_NODES // 2, _PAIR)
    wide_rows = _sc_gather(pairs, (flat_idx >> 1).reshape(-1))
    wide = wide_rows.reshape(n_tok, _K * _PAIR)

    out = _projection(wide, flat_idx, Wp, bp.reshape(1, -1))
    return out.reshape(bsz, seqlen, dmodel)
